# baseline (speedup 1.0000x reference)
import functools

import jax
import jax.numpy as jnp
from jax import lax
from jax.experimental import pallas as pl
from jax.experimental.pallas import tpu as pltpu


def _round_up(a, b):
    return (a + b - 1) // b * b


_NL = 64
_SHIFT = 6


def _scatter_body(nh, te, m_ref, idxr_ref, agg_ref):
    t = pl.program_id(1)

    @pl.when(t == 0)
    def _():
        agg_ref[...] = jnp.zeros_like(agg_ref)

    msg = jnp.tanh(m_ref[...] * 0.5)
    idx = idxr_ref[...]
    lo = idx & (_NL - 1)
    hi = idx >> _SHIFT
    lo_iota = lax.broadcasted_iota(jnp.int32, (_NL, te), 0)
    olo = (lo_iota == lo).astype(jnp.float32)
    b = jnp.concatenate([msg[0:1] * olo, msg[1:2] * olo], axis=0)
    hi_iota = lax.broadcasted_iota(jnp.int32, (nh, te), 0)
    ohi = (hi_iota == hi).astype(jnp.float32)
    agg_ref[...] += lax.dot_general(
        b, ohi, (((1,), (1,)), ((), ())),
        preferred_element_type=jnp.float32)[None]


def _gather_body(nh, te, m_ref, idxr_ref, aggp_ref, mlp_ref, out_ref):
    aggp = aggp_ref[...]
    mstack = aggp[0] + aggp[1]
    idx = idxr_ref[...]
    hi = idx >> _SHIFT
    lo = idx & (_NL - 1)
    hi_iota = lax.broadcasted_iota(jnp.int32, (nh, te), 0)
    ohi = (hi_iota == hi).astype(jnp.float32)
    r = jnp.dot(mstack, ohi, preferred_element_type=jnp.float32)
    lo_iota = lax.broadcasted_iota(jnp.int32, (_NL, te), 0)
    olo = (lo_iota == lo).astype(jnp.float32)
    msg = jnp.tanh(m_ref[...] * 0.5)
    g0 = jnp.sum(r[0:_NL] * olo, axis=0, keepdims=True)
    g1 = jnp.sum(r[_NL:2 * _NL] * olo, axis=0, keepdims=True)
    col0 = g0 - msg[0:1]
    col1 = g1 - msg[1:2]
    w1c = mlp_ref[:, 0:1]
    b1c = mlp_ref[:, 1:2]
    w2c = mlp_ref[:, 2:3]
    b2c = mlp_ref[0:1, 3:4]
    h = jnp.maximum(col0 * w1c + b1c, 0.0)
    y = jnp.sum(h * w2c, axis=0, keepdims=True) + b2c
    out_ref[...] = y * col1


def kernel(m, edge_index, x, w1, b1, w2, b2):
    m = m.astype(jnp.float32)
    E, F = m.shape
    N = x.shape[0] if x.ndim == 1 else x.shape[0]
    idx = edge_index[1].astype(jnp.int32)

    nh = _round_up(-(-N // _NL), 128)
    te = 16384
    e_pad = _round_up(max(E, 1), 2 * te)
    nt1 = e_pad // (2 * te)
    nt2 = e_pad // te

    m_lane = jnp.zeros((2, e_pad), jnp.float32).at[:, :E].set(m.T)
    idx_row = jnp.full((1, e_pad), nh * _NL, jnp.int32).at[0, :E].set(idx)

    hidden = w1.shape[1]
    mlp_packed = jnp.stack(
        [w1.reshape(-1).astype(jnp.float32),
         b1.reshape(-1).astype(jnp.float32),
         w2.reshape(-1).astype(jnp.float32),
         jnp.broadcast_to(b2.reshape(()).astype(jnp.float32), (hidden,))],
        axis=1)

    cparams = dict(vmem_limit_bytes=48 * 1024 * 1024)

    agg_partial = pl.pallas_call(
        functools.partial(_scatter_body, nh, te),
        out_shape=jax.ShapeDtypeStruct((2, 2 * _NL, nh), jnp.float32),
        grid_spec=pltpu.PrefetchScalarGridSpec(
            num_scalar_prefetch=0,
            grid=(2, nt1),
            in_specs=[pl.BlockSpec((2, te), lambda c, t: (0, c * nt1 + t)),
                      pl.BlockSpec((1, te), lambda c, t: (0, c * nt1 + t))],
            out_specs=pl.BlockSpec((1, 2 * _NL, nh), lambda c, t: (c, 0, 0)),
        ),
        compiler_params=pltpu.CompilerParams(
            dimension_semantics=("parallel", "arbitrary"), **cparams),
        cost_estimate=pl.CostEstimate(
            flops=2 * 2 * _NL * e_pad * nh,
            transcendentals=2 * e_pad,
            bytes_accessed=4 * (2 * e_pad + e_pad + 2 * 2 * _NL * nh)),
    )(m_lane, idx_row)

    out_lane = pl.pallas_call(
        functools.partial(_gather_body, nh, te),
        out_shape=jax.ShapeDtypeStruct((1, e_pad), jnp.float32),
        grid_spec=pltpu.PrefetchScalarGridSpec(
            num_scalar_prefetch=0,
            grid=(nt2,),
            in_specs=[pl.BlockSpec((2, te), lambda t: (0, t)),
                      pl.BlockSpec((1, te), lambda t: (0, t)),
                      pl.BlockSpec((2, 2 * _NL, nh), lambda t: (0, 0, 0)),
                      pl.BlockSpec((hidden, 4), lambda t: (0, 0))],
            out_specs=pl.BlockSpec((1, te), lambda t: (0, t)),
        ),
        compiler_params=pltpu.CompilerParams(
            dimension_semantics=("parallel",), **cparams),
        cost_estimate=pl.CostEstimate(
            flops=2 * 2 * _NL * nh * e_pad,
            transcendentals=2 * e_pad,
            bytes_accessed=4 * (2 * e_pad + e_pad + 2 * 2 * _NL * nh + e_pad)),
    )(m_lane, idx_row, agg_partial, mlp_packed)

    return out_lane[:, :E].T


# speedup vs baseline: 1.0069x; 1.0069x over previous
import functools

import jax
import jax.numpy as jnp
from jax import lax
from jax.experimental import pallas as pl
from jax.experimental.pallas import tpu as pltpu


def _round_up(a, b):
    return (a + b - 1) // b * b


_NL = 64
_SHIFT = 6


def _scatter_body(nh, te, sub, m_ref, idxr_ref, agg_ref):
    t = pl.program_id(1)

    @pl.when(t == 0)
    def _():
        agg_ref[...] = jnp.zeros_like(agg_ref)

    lo_iota = lax.broadcasted_iota(jnp.int32, (_NL, sub), 0)
    hi_iota = lax.broadcasted_iota(jnp.int32, (nh, sub), 0)
    acc = jnp.zeros((2 * _NL, nh), jnp.float32)
    for s in range(te // sub):
        sl = pl.ds(s * sub, sub)
        msg = jnp.tanh(m_ref[:, sl] * 0.5)
        idx = idxr_ref[:, sl]
        lo = idx & (_NL - 1)
        hi = idx >> _SHIFT
        olo = (lo_iota == lo).astype(jnp.float32)
        b = jnp.concatenate([msg[0:1] * olo, msg[1:2] * olo], axis=0)
        ohi = (hi_iota == hi).astype(jnp.float32)
        acc = acc + lax.dot_general(
            b, ohi, (((1,), (1,)), ((), ())),
            preferred_element_type=jnp.float32)
    agg_ref[...] += acc[None]


def _gather_body(nh, te, sub, m_ref, idxr_ref, aggp_ref, mlp_ref, out_ref):
    aggp = aggp_ref[...]
    mstack = aggp[0] + aggp[1]
    w1c = mlp_ref[:, 0:1]
    b1c = mlp_ref[:, 1:2]
    w2c = mlp_ref[:, 2:3]
    b2c = mlp_ref[0:1, 3:4]
    hi_iota = lax.broadcasted_iota(jnp.int32, (nh, sub), 0)
    lo_iota = lax.broadcasted_iota(jnp.int32, (_NL, sub), 0)
    for s in range(te // sub):
        sl = pl.ds(s * sub, sub)
        idx = idxr_ref[:, sl]
        hi = idx >> _SHIFT
        lo = idx & (_NL - 1)
        ohi = (hi_iota == hi).astype(jnp.float32)
        r = jnp.dot(mstack, ohi,
                    preferred_element_type=jnp.float32)
        olo = (lo_iota == lo).astype(jnp.float32)
        msg = jnp.tanh(m_ref[:, sl] * 0.5)
        g0 = jnp.sum(r[0:_NL] * olo, axis=0, keepdims=True)
        g1 = jnp.sum(r[_NL:2 * _NL] * olo, axis=0, keepdims=True)
        col0 = g0 - msg[0:1]
        col1 = g1 - msg[1:2]
        h = jnp.maximum(col0 * w1c + b1c, 0.0)
        y = jnp.sum(h * w2c, axis=0, keepdims=True) + b2c
        out_ref[:, sl] = y * col1


def kernel(m, edge_index, x, w1, b1, w2, b2):
    m = m.astype(jnp.float32)
    E, F = m.shape
    N = x.shape[0] if x.ndim == 1 else x.shape[0]
    idx = edge_index[1].astype(jnp.int32)

    nh = _round_up(-(-N // _NL), 128)
    te = 16384
    sub = 1024
    e_pad = _round_up(max(E, 1), 2 * te)
    nt1 = e_pad // (2 * te)
    nt2 = e_pad // te

    m_lane = jnp.zeros((2, e_pad), jnp.float32).at[:, :E].set(m.T)
    idx_row = jnp.full((1, e_pad), nh * _NL, jnp.int32).at[0, :E].set(idx)

    hidden = w1.shape[1]
    mlp_packed = jnp.stack(
        [w1.reshape(-1).astype(jnp.float32),
         b1.reshape(-1).astype(jnp.float32),
         w2.reshape(-1).astype(jnp.float32),
         jnp.broadcast_to(b2.reshape(()).astype(jnp.float32), (hidden,))],
        axis=1)

    cparams = dict(vmem_limit_bytes=48 * 1024 * 1024)

    agg_partial = pl.pallas_call(
        functools.partial(_scatter_body, nh, te, sub),
        out_shape=jax.ShapeDtypeStruct((2, 2 * _NL, nh), jnp.float32),
        grid_spec=pltpu.PrefetchScalarGridSpec(
            num_scalar_prefetch=0,
            grid=(2, nt1),
            in_specs=[pl.BlockSpec((2, te), lambda c, t: (0, c * nt1 + t)),
                      pl.BlockSpec((1, te), lambda c, t: (0, c * nt1 + t))],
            out_specs=pl.BlockSpec((1, 2 * _NL, nh), lambda c, t: (c, 0, 0)),
        ),
        compiler_params=pltpu.CompilerParams(
            dimension_semantics=("parallel", "arbitrary"), **cparams),
        cost_estimate=pl.CostEstimate(
            flops=2 * 2 * _NL * e_pad * nh,
            transcendentals=2 * e_pad,
            bytes_accessed=4 * (2 * e_pad + e_pad + 2 * 2 * _NL * nh)),
    )(m_lane, idx_row)

    out_lane = pl.pallas_call(
        functools.partial(_gather_body, nh, te, sub),
        out_shape=jax.ShapeDtypeStruct((1, e_pad), jnp.float32),
        grid_spec=pltpu.PrefetchScalarGridSpec(
            num_scalar_prefetch=0,
            grid=(nt2,),
            in_specs=[pl.BlockSpec((2, te), lambda t: (0, t)),
                      pl.BlockSpec((1, te), lambda t: (0, t)),
                      pl.BlockSpec((2, 2 * _NL, nh), lambda t: (0, 0, 0)),
                      pl.BlockSpec((hidden, 4), lambda t: (0, 0))],
            out_specs=pl.BlockSpec((1, te), lambda t: (0, t)),
        ),
        compiler_params=pltpu.CompilerParams(
            dimension_semantics=("parallel",), **cparams),
        cost_estimate=pl.CostEstimate(
            flops=2 * 2 * _NL * nh * e_pad,
            transcendentals=2 * e_pad,
            bytes_accessed=4 * (2 * e_pad + e_pad + 2 * 2 * _NL * nh + e_pad)),
    )(m_lane, idx_row, agg_partial, mlp_packed)

    return out_lane[:, :E].T
